# Initial kernel scaffold; baseline (speedup 1.0000x reference)
import functools

import jax
import jax.numpy as jnp
from jax import lax
from jax.experimental import pallas as pl
from jax.experimental.pallas import tpu as pltpu
from jax.experimental.pallas import tpu_sc as plsc


def _make_sc_gather(V, D, N):
    info = plsc.get_sparse_core_info()
    NC, NS = info.num_cores, info.num_subcores
    NW = NC * NS

    SEG = 128
    K = 8
    C = K * SEG
    per_w = N // NW
    n_chunks = per_w // C
    assert per_w % C == 0 and N % NW == 0

    mesh = plsc.VectorSubcoreMesh(core_axis_name="c", subcore_axis_name="s")

    @functools.partial(
        pl.kernel,
        mesh=mesh,
        out_type=jax.ShapeDtypeStruct((N, D), jnp.float32),
        scratch_types=[
            pltpu.VMEM((K, SEG), jnp.int32),
            pltpu.VMEM((C, D), jnp.float32),
            pltpu.SemaphoreType.DMA,
        ],
    )
    def grab(idx_hbm, table_hbm, out_hbm, idx_v, rows_v, sem):
        wid = lax.axis_index("s") * NC + lax.axis_index("c")

        def chunk_body(c, _):
            blk = (wid * n_chunks + c) * K
            pltpu.sync_copy(idx_hbm.at[pl.ds(blk, K)], idx_v)
            copies = []
            for j in range(K):
                copies.append(
                    pltpu.async_copy(
                        table_hbm.at[idx_v.at[j]],
                        rows_v.at[pl.ds(j * SEG, SEG)],
                        sem,
                    )
                )
            for cp in copies:
                cp.wait()
            base = (wid * n_chunks + c) * C
            pltpu.sync_copy(rows_v, out_hbm.at[pl.ds(base, C)])
            return ()

        lax.fori_loop(0, n_chunks, chunk_body, ())

    return grab


def kernel(input_seq, table):
    B, H = input_seq.shape
    V, D = table.shape
    N = B * H
    idx2d = input_seq.reshape(N // 128, 128).astype(jnp.int32)
    grab = _make_sc_gather(V, D, N)
    out = grab(idx2d, table)
    return out.reshape(B, H, D)


# speedup vs baseline: 4.8022x; 4.8022x over previous
import functools

import jax
import jax.numpy as jnp
from jax import lax
from jax.experimental import pallas as pl
from jax.experimental.pallas import tpu as pltpu
from jax.experimental.pallas import tpu_sc as plsc


def _make_sc_gather(V, D, N):
    info = plsc.get_sparse_core_info()
    NC, NS = info.num_cores, info.num_subcores
    NW = NC * NS

    SEG = 128
    K = 8
    C = K * SEG
    per_w = N // NW
    n_chunks = per_w // C
    assert per_w % C == 0 and N % NW == 0

    mesh = plsc.VectorSubcoreMesh(core_axis_name="c", subcore_axis_name="s")

    @functools.partial(
        pl.kernel,
        mesh=mesh,
        out_type=jax.ShapeDtypeStruct((N, D), jnp.float32),
        scratch_types=[
            pltpu.VMEM((K, SEG), jnp.int32),
            pltpu.VMEM((C, D), jnp.float32),
            pltpu.SemaphoreType.DMA,
        ],
        compiler_params=pltpu.CompilerParams(use_tc_tiling_on_sc=False),
    )
    def grab(idx_hbm, table_hbm, out_hbm, idx_v, rows_v, sem):
        wid = lax.axis_index("s") * NC + lax.axis_index("c")

        def chunk_body(c, _):
            blk = (wid * n_chunks + c) * K
            pltpu.sync_copy(idx_hbm.at[pl.ds(blk, K)], idx_v)
            copies = []
            for j in range(K):
                copies.append(
                    pltpu.async_copy(
                        table_hbm.at[idx_v.at[j]],
                        rows_v.at[pl.ds(j * SEG, SEG)],
                        sem,
                    )
                )
            for cp in copies:
                cp.wait()
            base = (wid * n_chunks + c) * C
            pltpu.sync_copy(rows_v, out_hbm.at[pl.ds(base, C)])
            return ()

        lax.fori_loop(0, n_chunks, chunk_body, ())

    return grab


def kernel(input_seq, table):
    B, H = input_seq.shape
    V, D = table.shape
    N = B * H
    idx2d = input_seq.reshape(N // 128, 128).astype(jnp.int32)
    grab = _make_sc_gather(V, D, N)
    out = grab(idx2d, table)
    return out.reshape(B, H, D)
